# baseline (speedup 1.0000x reference)
import functools

import jax
import jax.numpy as jnp
from jax import lax
from jax.experimental import pallas as pl
from jax.experimental.pallas import tpu as pltpu
from jax.experimental.pallas import tpu_sc as plsc

NU = 25000
PAD = 25600
TBL = 2 * PAD
D = 64
E = 800000
BATCH = 1024
NC, NS, L = 2, 16, 16
EPT = E // NS
CH = 80
CH2 = 2 * CH
STRIPE = PAD // NS
BLK = 2000
NB = EPT // BLK
CAP = BLK + CH2 + L

_mesh = plsc.VectorSubcoreMesh(core_axis_name="c", subcore_axis_name="s")
_sc_params = pltpu.CompilerParams(needs_layout_passes=False,
                                  use_tc_tiling_on_sc=False)


def _prop_body(x_hbm, src_hbm, dst_hbm, w_hbm, out_hbm,
               src_b, dst_b, w_b,
               src_c, idx_c, w_c,
               src_v0, src_v1, idx_v0, idx_v1, rows0, rows1,
               sem0, sem1, acc):
    c = lax.axis_index("c")
    s = lax.axis_index("s")

    def zrow(r, _):
        z = jnp.zeros((L,), jnp.float32)
        for q in range(D // L):
            rows0[r, pl.ds(q * L, L)] = z
        return 0
    lax.fori_loop(0, CH, zrow, 0)

    def zsrc(g, _):
        src_c[pl.ds(g * L, L)] = jnp.zeros((L,), jnp.int32)
        return 0
    lax.fori_loop(0, CAP // L, zsrc, 0)

    def zcp(k, _):
        pltpu.sync_copy(rows0, acc.at[pl.ds(s * STRIPE + k * CH, CH)])
        return 0
    lax.fori_loop(0, STRIPE // CH, zcp, 0)
    plsc.subcore_barrier()

    nu = jnp.int32(NU)
    base_node = c * nu
    lanes = lax.iota(jnp.int32, L)

    def idxcp(ck, src_v, idx_v):
        def body(i, _):
            sl_c = pl.ds(ck * CH + i * L, L)
            sl = pl.ds(i * L, L)
            src_v[sl] = src_c[sl_c]
            idx_v[sl] = idx_c[sl_c]
            return 0
        lax.fori_loop(0, CH // L, body, 0)

    def scale_scatter(ck, rows, idx_v):
        wbase = ck * CH

        def body(jj, _):
            for u in range(4):
                j = jj * 4 + u
                wspl = plsc.load_gather(
                    w_c, [jnp.zeros((L,), jnp.int32) + (wbase + j)])
                for q in range(D // L):
                    sl = pl.ds(q * L, L)
                    rows[j, sl] = rows[j, sl] * wspl
            return 0
        lax.fori_loop(0, CH // 4, body, 0)
        pltpu.sync_copy(rows, acc.at[idx_v], add=True)

    def block(b, _):
        base = s * EPT + b * BLK
        pltpu.sync_copy(src_hbm.at[pl.ds(base, BLK)], src_b)
        pltpu.sync_copy(dst_hbm.at[pl.ds(base, BLK)], dst_b)
        pltpu.sync_copy(w_hbm.at[pl.ds(base, BLK)], w_b)

        def grp(g, cnt):
            sl = pl.ds(g * L, L)
            sv = src_b[sl]
            dv = dst_b[sl]
            wv = w_b[sl]
            ld = dv - base_node
            ok = (ld >= 0) & (ld < nu)
            svp = jnp.where(sv >= nu, sv + (PAD - NU), sv)
            plsc.store_compressed(src_c.at[pl.ds(cnt, L)], svp, mask=ok)
            plsc.store_compressed(idx_c.at[pl.ds(cnt, L)], ld, mask=ok)
            plsc.store_compressed(w_c.at[pl.ds(cnt, L)], wv, mask=ok)
            return cnt + jnp.sum(ok.astype(jnp.int32))
        cnt = lax.fori_loop(0, BLK // L, grp, jnp.int32(0))

        cnt_pad = (cnt + CH2 - 1) // CH2 * CH2

        def padg(g, _):
            sl = pl.ds(g * L, L)
            gl = lanes + g * L
            m = gl >= cnt
            idx_c[sl] = jnp.where(m, nu + (gl & 511), idx_c[sl])
            return 0
        lax.fori_loop(cnt // L, cnt_pad // L, padg, 0)

        nch = cnt_pad // CH

        @pl.when(nch > 0)
        def _():
            idxcp(0, src_v0, idx_v0)
            pltpu.async_copy(x_hbm.at[src_v0], rows0, sem0)

        def pair(j, _):
            idxcp(2 * j + 1, src_v1, idx_v1)
            pltpu.async_copy(x_hbm.at[src_v1], rows1, sem1)
            pltpu.make_async_copy(x_hbm.at[src_v0], rows0, sem0).wait()
            scale_scatter(2 * j, rows0, idx_v0)

            @pl.when(2 * j + 2 < nch)
            def _():
                idxcp(2 * j + 2, src_v0, idx_v0)
                pltpu.async_copy(x_hbm.at[src_v0], rows0, sem0)
            pltpu.make_async_copy(x_hbm.at[src_v1], rows1, sem1).wait()
            scale_scatter(2 * j + 1, rows1, idx_v1)
            return 0
        lax.fori_loop(0, nch // 2, pair, 0)
        return 0

    lax.fori_loop(0, NB, block, 0)
    plsc.subcore_barrier()

    pltpu.sync_copy(acc.at[pl.ds(s * STRIPE, STRIPE)],
                    out_hbm.at[pl.ds(c * PAD + s * STRIPE, STRIPE)])


_propagate = pl.kernel(
    _prop_body,
    out_type=jax.ShapeDtypeStruct((TBL, D), jnp.float32),
    mesh=_mesh,
    compiler_params=_sc_params,
    scratch_types=[
        pltpu.VMEM((BLK,), jnp.int32),
        pltpu.VMEM((BLK,), jnp.int32),
        pltpu.VMEM((BLK,), jnp.float32),
        pltpu.VMEM((CAP,), jnp.int32),
        pltpu.VMEM((CAP,), jnp.int32),
        pltpu.VMEM((CAP,), jnp.float32),
        pltpu.VMEM((CH,), jnp.int32),
        pltpu.VMEM((CH,), jnp.int32),
        pltpu.VMEM((CH,), jnp.int32),
        pltpu.VMEM((CH,), jnp.int32),
        pltpu.VMEM((CH, D), jnp.float32),
        pltpu.VMEM((CH, D), jnp.float32),
        pltpu.SemaphoreType.DMA,
        pltpu.SemaphoreType.DMA,
        pltpu.VMEM_SHARED((PAD, D), jnp.float32),
    ],
)

UPW = BATCH // (NC * NS)


def _gusers_body(u_hbm, x0, x1, x2, x3, out_hbm, uidx_v, a_v, b_v):
    c = lax.axis_index("c")
    s = lax.axis_index("s")
    w = s * NC + c
    base = w * UPW
    pltpu.sync_copy(u_hbm.at[pl.ds(base, UPW)], uidx_v)
    pltpu.sync_copy(x0.at[uidx_v], a_v)
    for t in (x1, x2, x3):
        pltpu.sync_copy(t.at[uidx_v], b_v)

        def addr(r, _):
            for q in range(D // L):
                sl = pl.ds(q * L, L)
                a_v[r, sl] = a_v[r, sl] + b_v[r, sl]
            return 0
        lax.fori_loop(0, UPW, addr, 0)
    pltpu.sync_copy(a_v, out_hbm.at[pl.ds(base, UPW)])


_gather_users = pl.kernel(
    _gusers_body,
    out_type=jax.ShapeDtypeStruct((BATCH, D), jnp.float32),
    mesh=_mesh,
    compiler_params=_sc_params,
    scratch_types=[
        pltpu.VMEM((UPW,), jnp.int32),
        pltpu.VMEM((UPW, D), jnp.float32),
        pltpu.VMEM((UPW, D), jnp.float32),
    ],
)

BN = 512


def _mm_body(u_ref, i0, i1, i2, i3, o_ref):
    items = i0[...] + i1[...] + i2[...] + i3[...]
    acc = lax.dot_general(u_ref[...], items, (((1,), (1,)), ((), ())),
                          preferred_element_type=jnp.float32)
    o_ref[...] = jax.nn.sigmoid(acc * 0.0625)


def _rating(u_s, x0, x1, x2, x3):
    item_spec = pl.BlockSpec((BN, D), lambda j: (PAD // BN + j, 0))
    return pl.pallas_call(
        _mm_body,
        grid=(PAD // BN,),
        in_specs=[pl.BlockSpec((BATCH, D), lambda j: (0, 0)),
                  item_spec, item_spec, item_spec, item_spec],
        out_specs=pl.BlockSpec((BATCH, BN), lambda j: (0, j)),
        out_shape=jax.ShapeDtypeStruct((BATCH, PAD), jnp.float32),
    )(u_s, x0, x1, x2, x3)


@jax.jit
def kernel(users, edge_index, edge_weight, user_emb, item_emb):
    users = users.astype(jnp.int32)
    src = edge_index[0].astype(jnp.int32)
    dst = edge_index[1].astype(jnp.int32)
    w = edge_weight.astype(jnp.float32)

    x0 = jnp.zeros((TBL, D), jnp.float32)
    x0 = x0.at[:NU].set(user_emb).at[PAD:PAD + NU].set(item_emb)

    x1 = _propagate(x0, src, dst, w)
    x2 = _propagate(x1, src, dst, w)
    x3 = _propagate(x2, src, dst, w)

    u_s = _gather_users(users, x0, x1, x2, x3)
    return _rating(u_s, x0, x1, x2, x3)[:, :NU]


# speedup vs baseline: 2.0659x; 2.0659x over previous
import functools

import jax
import jax.numpy as jnp
from jax import lax
from jax.experimental import pallas as pl
from jax.experimental.pallas import tpu as pltpu
from jax.experimental.pallas import tpu_sc as plsc

NU = 25000
PAD = 25600
TBL = 2 * PAD
D = 64
E = 800000
BATCH = 1024
NC, NS, L = 2, 16, 16
EPT = E // NS
CH = 80
STRIPE = PAD // NS
BLK = 2000
CPB = BLK // CH
NB = EPT // BLK

_mesh = plsc.VectorSubcoreMesh(core_axis_name="c", subcore_axis_name="s")
_sc_params = pltpu.CompilerParams(needs_layout_passes=False,
                                  use_tc_tiling_on_sc=False)


def _prop_body(x_hbm, src_hbm, dst_hbm, w_hbm, out_hbm,
               src_b, dst_b, w_b,
               src_v0, src_v1, idx_v0, idx_v1, rows0, rows1,
               sem0, sem1, acc):
    c = lax.axis_index("c")
    s = lax.axis_index("s")

    def zrow(r, _):
        z = jnp.zeros((L,), jnp.float32)
        for q in range(D // L):
            rows0[r, pl.ds(q * L, L)] = z
        return 0
    lax.fori_loop(0, CH, zrow, 0)

    def zcp(k, _):
        pltpu.sync_copy(rows0, acc.at[pl.ds(s * STRIPE + k * CH, CH)])
        return 0
    lax.fori_loop(0, STRIPE // CH, zcp, 0)
    plsc.subcore_barrier()

    nu = jnp.int32(NU)
    base_node = c * nu

    def remap(ck, src_v, idx_v):
        def body(i, _):
            sl_b = pl.ds(ck * CH + i * L, L)
            sl = pl.ds(i * L, L)
            sv = src_b[sl_b]
            src_v[sl] = jnp.where(sv >= nu, sv + (PAD - NU), sv)
            dv = dst_b[sl_b]
            ld = dv - base_node
            ok = (ld >= 0) & (ld < nu)
            idx_v[sl] = jnp.where(ok, ld, nu + (dv & 511))
            return 0
        lax.fori_loop(0, CH // L, body, 0)

    def scale_scatter(ck, rows, idx_v):
        wbase = ck * CH

        def body(jj, _):
            for u in range(4):
                j = jj * 4 + u
                wspl = plsc.load_gather(
                    w_b, [jnp.zeros((L,), jnp.int32) + (wbase + j)])
                for q in range(D // L):
                    sl = pl.ds(q * L, L)
                    rows[j, sl] = rows[j, sl] * wspl
            return 0
        lax.fori_loop(0, CH // 4, body, 0)
        pltpu.sync_copy(rows, acc.at[idx_v], add=True)

    def block(b, _):
        base = s * EPT + b * BLK
        pltpu.sync_copy(src_hbm.at[pl.ds(base, BLK)], src_b)
        pltpu.sync_copy(dst_hbm.at[pl.ds(base, BLK)], dst_b)
        pltpu.sync_copy(w_hbm.at[pl.ds(base, BLK)], w_b)

        remap(0, src_v0, idx_v0)
        pltpu.async_copy(x_hbm.at[src_v0], rows0, sem0)

        def pair(j, _):
            remap(2 * j + 1, src_v1, idx_v1)
            pltpu.async_copy(x_hbm.at[src_v1], rows1, sem1)
            pltpu.make_async_copy(x_hbm.at[src_v0], rows0, sem0).wait()
            scale_scatter(2 * j, rows0, idx_v0)
            remap(2 * j + 2, src_v0, idx_v0)
            pltpu.async_copy(x_hbm.at[src_v0], rows0, sem0)
            pltpu.make_async_copy(x_hbm.at[src_v1], rows1, sem1).wait()
            scale_scatter(2 * j + 1, rows1, idx_v1)
            return 0
        lax.fori_loop(0, (CPB - 1) // 2, pair, 0)

        pltpu.make_async_copy(x_hbm.at[src_v0], rows0, sem0).wait()
        scale_scatter(CPB - 1, rows0, idx_v0)
        return 0

    lax.fori_loop(0, NB, block, 0)
    plsc.subcore_barrier()

    pltpu.sync_copy(acc.at[pl.ds(s * STRIPE, STRIPE)],
                    out_hbm.at[pl.ds(c * PAD + s * STRIPE, STRIPE)])


_propagate = pl.kernel(
    _prop_body,
    out_type=jax.ShapeDtypeStruct((TBL, D), jnp.float32),
    mesh=_mesh,
    compiler_params=_sc_params,
    scratch_types=[
        pltpu.VMEM((BLK,), jnp.int32),
        pltpu.VMEM((BLK,), jnp.int32),
        pltpu.VMEM((BLK,), jnp.float32),
        pltpu.VMEM((CH,), jnp.int32),
        pltpu.VMEM((CH,), jnp.int32),
        pltpu.VMEM((CH,), jnp.int32),
        pltpu.VMEM((CH,), jnp.int32),
        pltpu.VMEM((CH, D), jnp.float32),
        pltpu.VMEM((CH, D), jnp.float32),
        pltpu.SemaphoreType.DMA,
        pltpu.SemaphoreType.DMA,
        pltpu.VMEM_SHARED((PAD, D), jnp.float32),
    ],
)

UPW = BATCH // (NC * NS)


def _gusers_body(u_hbm, x0, x1, x2, x3, out_hbm, uidx_v, a_v, b_v):
    c = lax.axis_index("c")
    s = lax.axis_index("s")
    w = s * NC + c
    base = w * UPW
    pltpu.sync_copy(u_hbm.at[pl.ds(base, UPW)], uidx_v)
    pltpu.sync_copy(x0.at[uidx_v], a_v)
    for t in (x1, x2, x3):
        pltpu.sync_copy(t.at[uidx_v], b_v)

        def addr(r, _):
            for q in range(D // L):
                sl = pl.ds(q * L, L)
                a_v[r, sl] = a_v[r, sl] + b_v[r, sl]
            return 0
        lax.fori_loop(0, UPW, addr, 0)
    pltpu.sync_copy(a_v, out_hbm.at[pl.ds(base, UPW)])


_gather_users = pl.kernel(
    _gusers_body,
    out_type=jax.ShapeDtypeStruct((BATCH, D), jnp.float32),
    mesh=_mesh,
    compiler_params=_sc_params,
    scratch_types=[
        pltpu.VMEM((UPW,), jnp.int32),
        pltpu.VMEM((UPW, D), jnp.float32),
        pltpu.VMEM((UPW, D), jnp.float32),
    ],
)

SUMB = 1024


def _sum_body(a, b, c, d, o):
    o[...] = a[...] + b[...] + c[...] + d[...]


def _sum_items(x0, x1, x2, x3):
    in_spec = pl.BlockSpec((SUMB, D), lambda j: (PAD // SUMB + j, 0))
    return pl.pallas_call(
        _sum_body,
        grid=(PAD // SUMB,),
        in_specs=[in_spec, in_spec, in_spec, in_spec],
        out_specs=pl.BlockSpec((SUMB, D), lambda j: (j, 0)),
        out_shape=jax.ShapeDtypeStruct((PAD, D), jnp.float32),
    )(x0, x1, x2, x3)


MB = 64


def _mm_body(u_ref, it_ref, o_ref):
    acc = lax.dot_general(u_ref[...], it_ref[:NU, :],
                          (((1,), (1,)), ((), ())),
                          preferred_element_type=jnp.float32)
    o_ref[...] = jax.nn.sigmoid(acc * 0.0625)


def _rating(u_s, items_sum):
    return pl.pallas_call(
        _mm_body,
        grid=(BATCH // MB,),
        in_specs=[pl.BlockSpec((MB, D), lambda j: (j, 0)),
                  pl.BlockSpec((PAD, D), lambda j: (0, 0))],
        out_specs=pl.BlockSpec((MB, NU), lambda j: (j, 0)),
        out_shape=jax.ShapeDtypeStruct((BATCH, NU), jnp.float32),
    )(u_s, items_sum)


@jax.jit
def kernel(users, edge_index, edge_weight, user_emb, item_emb):
    users = users.astype(jnp.int32)
    src = edge_index[0].astype(jnp.int32)
    dst = edge_index[1].astype(jnp.int32)
    w = edge_weight.astype(jnp.float32)

    x0 = jnp.zeros((TBL, D), jnp.float32)
    x0 = x0.at[:NU].set(user_emb).at[PAD:PAD + NU].set(item_emb)

    x1 = _propagate(x0, src, dst, w)
    x2 = _propagate(x1, src, dst, w)
    x3 = _propagate(x2, src, dst, w)

    i_sum = _sum_items(x0, x1, x2, x3)
    u_s = _gather_users(users, x0, x1, x2, x3)
    return _rating(u_s, i_sum)
